# baseline (speedup 1.0000x reference)
import functools

import jax
import jax.numpy as jnp
from jax import lax
from jax.experimental import pallas as pl
from jax.experimental.pallas import tpu as pltpu
from jax.experimental.pallas import tpu_sc as plsc

N = 10000
E = 320000
D = 128
H = 128
H2 = 64
DT = 16
G = 128

NC = 2
NS = 16
NW = NC * NS
EPW = E // NW
CH = 40
NCH = EPW // CH
RPT = N // NS
LB = 5
GA = 2

BR = 2000
NB = N // BR

EPS = 1e-5


def _sc_segment_sum(x, src_r, dst_r, zeros):
    mesh = plsc.VectorSubcoreMesh(core_axis_name="c", subcore_axis_name="s")

    @functools.partial(
        pl.kernel,
        out_type=jax.ShapeDtypeStruct((NC * N, D), jnp.float32),
        mesh=mesh,
        compiler_params=pltpu.CompilerParams(use_tc_tiling_on_sc=False),
        scratch_types=[
            pltpu.VMEM((NCH, CH), jnp.int32),
            pltpu.VMEM((NCH, CH), jnp.int32),
            pltpu.VMEM((LB, CH, D), jnp.float32),
            pltpu.VMEM_SHARED((N, D), jnp.float32),
        ] + [pltpu.SemaphoreType.DMA] * (2 * LB),
    )
    def k(x_hbm, src_hbm, dst_hbm, z_hbm, out_hbm, src_v, dst_v, rows_v, acc,
          *sems):
        gsem = sems[:LB]
        ssem = sems[LB:]
        cid = lax.axis_index("c")
        sid = lax.axis_index("s")
        wid = cid * NS + sid
        pltpu.sync_copy(z_hbm.at[pl.ds(sid * RPT, RPT)],
                        acc.at[pl.ds(sid * RPT, RPT)])
        pltpu.sync_copy(src_hbm.at[pl.ds(wid * NCH, NCH)], src_v)
        pltpu.sync_copy(dst_hbm.at[pl.ds(wid * NCH, NCH)], dst_v)
        plsc.subcore_barrier()

        def gstart(j, b):
            pltpu.async_copy(x_hbm.at[src_v.at[j]], rows_v.at[b], gsem[b])

        def gwait(j, b):
            pltpu.make_async_copy(x_hbm.at[src_v.at[j]], rows_v.at[b],
                                  gsem[b]).wait()

        def sstart(j, b):
            pltpu.async_copy(rows_v.at[b], acc.at[dst_v.at[j]], ssem[b],
                             add=True)

        def swait(j, b):
            pltpu.make_async_copy(rows_v.at[b], acc.at[dst_v.at[j]],
                                  ssem[b]).wait()

        for b in range(LB):
            gstart(b, b)
        for b in range(LB - GA):
            gwait(b, b)
            sstart(b, b)

        def body(t, carry):
            base = LB * t
            for b in range(LB):
                j = base + b
                swait(j - LB, b)
                gstart(j, b)
                bg = (b - GA) % LB
                gwait(j - GA, bg)
                sstart(j - GA, bg)
            return carry

        lax.fori_loop(1, NCH // LB, body, 0)
        for j in range(LB * (NCH // LB), NCH):
            b = j % LB
            swait(j - LB, b)
            gstart(j, b)
            gwait(j - GA, (j - GA) % LB)
            sstart(j - GA, (j - GA) % LB)
        for j in range(NCH, NCH + GA):
            swait(j - LB, (j - LB) % LB)
            gwait(j - GA, (j - GA) % LB)
            sstart(j - GA, (j - GA) % LB)
        for j in range(NCH + GA, NCH + LB):
            swait(j - LB, (j - LB) % LB)
        plsc.subcore_barrier()
        pltpu.sync_copy(acc.at[pl.ds(sid * RPT, RPT)],
                        out_hbm.at[pl.ds(cid * N + sid * RPT, RPT)])

    return k(x, src_r, dst_r, zeros)


def _elu(y):
    return jnp.where(y > 0, y, jnp.exp(jnp.minimum(y, 0.0)) - 1.0)


def _bn(y, s, g, be):
    m = s[0:1, :] * (1.0 / N)
    v = s[1:2, :] * (1.0 / N) - m * m
    return (y - m) * lax.rsqrt(v + EPS) * g + be


def _moments(y):
    s = jnp.concatenate(
        [jnp.sum(y, axis=0, keepdims=True),
         jnp.sum(y * y, axis=0, keepdims=True)], axis=0)
    return jnp.concatenate([s, jnp.zeros((6, y.shape[1]), jnp.float32)],
                           axis=0)


def _acc_add(ref, val, first):
    @pl.when(first)
    def _():
        ref[...] = val

    @pl.when(jnp.logical_not(first))
    def _():
        ref[...] = ref[...] + val


def _acc_max(ref, val, first):
    @pl.when(first)
    def _():
        ref[...] = val

    @pl.when(jnp.logical_not(first))
    def _():
        ref[...] = jnp.maximum(ref[...], val)


def _segmax(z, mask):
    cols = []
    for c in range(DT):
        mc = jnp.max(jnp.where(mask, z[:, c:c + 1], -jnp.inf), axis=0,
                     keepdims=True)
        cols.append(mc)
    return jnp.concatenate(cols, axis=0)


def _full(shape):
    return pl.BlockSpec(shape, lambda s: (0, 0))


def _blkA(w):
    return pl.BlockSpec((BR, w), lambda s: (jnp.minimum(s, NB - 1), 0))


def _blkB(w):
    return pl.BlockSpec((BR, w), lambda s: (jnp.maximum(s - NB, 0), 0))


def _mask_of(batch_blk):
    gids = lax.broadcasted_iota(jnp.int32, (1, G), 1)
    return batch_blk == gids


def _k1(x_ref, aa_ref, ab_ref, W0_ref, b0_ref, Wc_ref, bc_ref,
        g0_ref, be0_ref, gc_ref, bec_ref, W1_ref, b1_ref, batch_ref,
        z0_ref, x1_ref, yz_ref, sz_ref, m0_ref,
        y0s, y1s, s0a, s1a, sza, m0a):
    s = pl.program_id(0)

    @pl.when(s < NB)
    def _():
        x = x_ref[...]
        y0 = jnp.dot(x, W0_ref[...],
                     preferred_element_type=jnp.float32) + b0_ref[...]
        xin = x + aa_ref[...] + ab_ref[...]
        y1 = jnp.dot(xin, Wc_ref[...],
                     preferred_element_type=jnp.float32) + bc_ref[...]
        y0s[pl.ds(s * BR, BR), :] = y0
        y1s[pl.ds(s * BR, BR), :] = y1
        _acc_add(s0a, _moments(y0), s == 0)
        _acc_add(s1a, _moments(y1), s == 0)

    @pl.when(s >= NB)
    def _():
        i = s - NB
        y0 = y0s[pl.ds(i * BR, BR), :]
        y1 = y1s[pl.ds(i * BR, BR), :]
        z0 = _elu(_bn(y0, s0a[...], g0_ref[...], be0_ref[...]))
        x1 = _elu(_bn(y1, s1a[...], gc_ref[...], bec_ref[...]))
        z0_ref[...] = z0
        x1_ref[...] = x1
        yz = jnp.dot(x1, W1_ref[...],
                     preferred_element_type=jnp.float32) + b1_ref[...]
        yz_ref[...] = yz
        _acc_add(sza, _moments(yz), i == 0)
        _acc_max(m0a, _segmax(z0, _mask_of(batch_ref[...])), i == 0)

    @pl.when(s == 2 * NB - 1)
    def _():
        sz_ref[...] = sza[...]
        m0_ref[...] = m0a[...]


def _layer1(x, aggp, W0, b0, Wc, bc, g0, be0, gc, bec, W1, b1, batch2):
    return pl.pallas_call(
        _k1,
        grid=(2 * NB,),
        in_specs=[
            _blkA(D),
            pl.BlockSpec((BR, D), lambda s: (jnp.minimum(s, NB - 1), 0)),
            pl.BlockSpec((BR, D), lambda s: (jnp.minimum(s, NB - 1) + NB, 0)),
            _full((D, DT)), _full((1, DT)), _full((D, H)), _full((1, H)),
            _full((1, DT)), _full((1, DT)), _full((1, H)), _full((1, H)),
            _full((H, DT)), _full((1, DT)),
            pl.BlockSpec((BR, 1), lambda s: (jnp.maximum(s - NB, 0), 0)),
        ],
        out_specs=[_blkB(DT), _blkB(H), _blkB(DT), _full((8, DT)),
                   _full((DT, G))],
        out_shape=[
            jax.ShapeDtypeStruct((N, DT), jnp.float32),
            jax.ShapeDtypeStruct((N, H), jnp.float32),
            jax.ShapeDtypeStruct((N, DT), jnp.float32),
            jax.ShapeDtypeStruct((8, DT), jnp.float32),
            jax.ShapeDtypeStruct((DT, G), jnp.float32),
        ],
        scratch_shapes=[
            pltpu.VMEM((N, DT), jnp.float32),
            pltpu.VMEM((N, H), jnp.float32),
            pltpu.VMEM((8, DT), jnp.float32),
            pltpu.VMEM((8, H), jnp.float32),
            pltpu.VMEM((8, DT), jnp.float32),
            pltpu.VMEM((DT, G), jnp.float32),
        ],
    )(x, aggp, aggp, W0, b0, Wc, bc, g0, be0, gc, bec, W1, b1, batch2)


def _k2(yz_ref, sz_ref, g1_ref, be1_ref, batch_ref, z0_ref,
        zp_ref, m1_ref, z1s, esa, m1a):
    s = pl.program_id(0)

    @pl.when(s < NB)
    def _():
        z1 = _elu(_bn(yz_ref[...], sz_ref[...], g1_ref[...], be1_ref[...]))
        z1s[pl.ds(s * BR, BR), :] = z1
        e = jnp.sum(jnp.exp(z1), axis=0, keepdims=True)
        e = jnp.concatenate([e, jnp.zeros((7, DT), jnp.float32)], axis=0)
        _acc_add(esa, e, s == 0)
        _acc_max(m1a, _segmax(z1, _mask_of(batch_ref[...])), s == 0)

    @pl.when(s >= NB)
    def _():
        i = s - NB
        z1 = z1s[pl.ds(i * BR, BR), :]
        zp_ref[...] = z0_ref[...] + (jnp.exp(z1) / esa[0:1, :]) * z1

    @pl.when(s == 2 * NB - 1)
    def _():
        m1_ref[...] = m1a[...]


def _layer2(yz, sz, g1, be1, batch2, z0):
    return pl.pallas_call(
        _k2,
        grid=(2 * NB,),
        in_specs=[
            _blkA(DT), _full((8, DT)), _full((1, DT)), _full((1, DT)),
            pl.BlockSpec((BR, 1), lambda s: (jnp.minimum(s, NB - 1), 0)),
            _blkB(DT),
        ],
        out_specs=[_blkB(DT), _full((DT, G))],
        out_shape=[
            jax.ShapeDtypeStruct((N, DT), jnp.float32),
            jax.ShapeDtypeStruct((DT, G), jnp.float32),
        ],
        scratch_shapes=[
            pltpu.VMEM((N, DT), jnp.float32),
            pltpu.VMEM((8, DT), jnp.float32),
            pltpu.VMEM((DT, G), jnp.float32),
        ],
    )(yz, sz, g1, be1, batch2, z0)


def _k3(x1_ref, aa_ref, ab_ref, Wc_ref, bc_ref, gc_ref, bec_ref,
        W2_ref, b2_ref,
        x2_ref, yz_ref, sz_ref, y2s, s2a, sza):
    s = pl.program_id(0)

    @pl.when(s < NB)
    def _():
        xin = x1_ref[...] + aa_ref[...] + ab_ref[...]
        y2 = jnp.dot(xin, Wc_ref[...],
                     preferred_element_type=jnp.float32) + bc_ref[...]
        y2s[pl.ds(s * BR, BR), :] = y2
        _acc_add(s2a, _moments(y2), s == 0)

    @pl.when(s >= NB)
    def _():
        i = s - NB
        y2 = y2s[pl.ds(i * BR, BR), :]
        x2 = _elu(_bn(y2, s2a[...], gc_ref[...], bec_ref[...]))
        x2_ref[...] = x2
        yz = jnp.dot(x2, W2_ref[...],
                     preferred_element_type=jnp.float32) + b2_ref[...]
        yz_ref[...] = yz
        _acc_add(sza, _moments(yz), i == 0)

    @pl.when(s == 2 * NB - 1)
    def _():
        sz_ref[...] = sza[...]


def _layer3(x1, aggp, Wc, bc, gc, bec, W2, b2):
    return pl.pallas_call(
        _k3,
        grid=(2 * NB,),
        in_specs=[
            _blkA(D),
            pl.BlockSpec((BR, D), lambda s: (jnp.minimum(s, NB - 1), 0)),
            pl.BlockSpec((BR, D), lambda s: (jnp.minimum(s, NB - 1) + NB, 0)),
            _full((D, H2)), _full((1, H2)), _full((1, H2)), _full((1, H2)),
            _full((H2, DT)), _full((1, DT)),
        ],
        out_specs=[_blkB(H2), _blkB(DT), _full((8, DT))],
        out_shape=[
            jax.ShapeDtypeStruct((N, H2), jnp.float32),
            jax.ShapeDtypeStruct((N, DT), jnp.float32),
            jax.ShapeDtypeStruct((8, DT), jnp.float32),
        ],
        scratch_shapes=[
            pltpu.VMEM((N, H2), jnp.float32),
            pltpu.VMEM((8, H2), jnp.float32),
            pltpu.VMEM((8, DT), jnp.float32),
        ],
    )(x1, aggp, aggp, Wc, bc, gc, bec, W2, b2)


def _k4(yz_ref, sz_ref, g2_ref, be2_ref, batch_ref, zp_ref, m0_ref, m1_ref,
        z_ref, ot_ref, z2s, esa, m2a):
    s = pl.program_id(0)

    @pl.when(s < NB)
    def _():
        z2 = _elu(_bn(yz_ref[...], sz_ref[...], g2_ref[...], be2_ref[...]))
        z2s[pl.ds(s * BR, BR), :] = z2
        e = jnp.sum(jnp.exp(z2), axis=0, keepdims=True)
        e = jnp.concatenate([e, jnp.zeros((7, DT), jnp.float32)], axis=0)
        _acc_add(esa, e, s == 0)
        _acc_max(m2a, _segmax(z2, _mask_of(batch_ref[...])), s == 0)

    @pl.when(s >= NB)
    def _():
        i = s - NB
        z2 = z2s[pl.ds(i * BR, BR), :]
        z_ref[...] = zp_ref[...] + (jnp.exp(z2) / esa[0:1, :]) * z2

    @pl.when(s == 2 * NB - 1)
    def _():
        ot_ref[...] = m0_ref[...] + m1_ref[...] + m2a[...]


def _layer4(yz2, sz2, g2, be2, batch2, zp, m0, m1):
    return pl.pallas_call(
        _k4,
        grid=(2 * NB,),
        in_specs=[
            _blkA(DT), _full((8, DT)), _full((1, DT)), _full((1, DT)),
            pl.BlockSpec((BR, 1), lambda s: (jnp.minimum(s, NB - 1), 0)),
            _blkB(DT), _full((DT, G)), _full((DT, G)),
        ],
        out_specs=[_blkB(DT), _full((DT, G))],
        out_shape=[
            jax.ShapeDtypeStruct((N, DT), jnp.float32),
            jax.ShapeDtypeStruct((DT, G), jnp.float32),
        ],
        scratch_shapes=[
            pltpu.VMEM((N, DT), jnp.float32),
            pltpu.VMEM((8, DT), jnp.float32),
            pltpu.VMEM((DT, G), jnp.float32),
        ],
    )(yz2, sz2, g2, be2, batch2, zp, m0, m1)


def kernel(x, edge_index, batch, W0, b0, g0, be0, Wc1, bc1, gc1, bec1,
           W1, b1, g1, be1, Wc2, bc2, gc2, bec2, W2, b2, g2, be2):
    src_r = edge_index[0].reshape(NW * NCH, CH)
    dst_r = edge_index[1].reshape(NW * NCH, CH)
    zeros = jnp.zeros((N, D), jnp.float32)
    batch2 = batch.reshape(N, 1)
    r2 = lambda a: a.reshape(1, -1)

    agg1p = _sc_segment_sum(x, src_r, dst_r, zeros)
    z0, x1, yz1, sz1, m0 = _layer1(x, agg1p, W0, r2(b0), Wc1, r2(bc1),
                                   r2(g0), r2(be0), r2(gc1), r2(bec1),
                                   W1, r2(b1), batch2)
    agg2p = _sc_segment_sum(x1, src_r, dst_r, zeros)
    Zp, m1 = _layer2(yz1, sz1, r2(g1), r2(be1), batch2, z0)
    x2, yz2, sz2 = _layer3(x1, agg2p, Wc2, r2(bc2), r2(gc2), r2(bec2),
                           W2, r2(b2))
    Z, out_t = _layer4(yz2, sz2, r2(g2), r2(be2), batch2, Zp, m0, m1)
    return out_t.T, Z, x2


# speedup vs baseline: 1.0552x; 1.0552x over previous
import functools

import jax
import jax.numpy as jnp
from jax import lax
from jax.experimental import pallas as pl
from jax.experimental.pallas import tpu as pltpu
from jax.experimental.pallas import tpu_sc as plsc

N = 10000
E = 320000
D = 128
H = 128
H2 = 64
DT = 16
G = 128

NC = 2
NS = 16
NW = NC * NS
EPW = E // NW
CH = 40
NCH = EPW // CH
RPT = N // NS
LB = 5
GA = 3

BR = 2000
NB = N // BR

EPS = 1e-5


def _sc_segment_sum(x, src_r, dst_r, zeros):
    mesh = plsc.VectorSubcoreMesh(core_axis_name="c", subcore_axis_name="s")

    @functools.partial(
        pl.kernel,
        out_type=jax.ShapeDtypeStruct((NC * N, D), jnp.float32),
        mesh=mesh,
        compiler_params=pltpu.CompilerParams(use_tc_tiling_on_sc=False),
        scratch_types=[
            pltpu.VMEM((NCH, CH), jnp.int32),
            pltpu.VMEM((NCH, CH), jnp.int32),
            pltpu.VMEM((LB, CH, D), jnp.float32),
            pltpu.VMEM_SHARED((N, D), jnp.float32),
        ] + [pltpu.SemaphoreType.DMA] * (2 * LB),
    )
    def k(x_hbm, src_hbm, dst_hbm, z_hbm, out_hbm, src_v, dst_v, rows_v, acc,
          *sems):
        gsem = sems[:LB]
        ssem = sems[LB:]
        cid = lax.axis_index("c")
        sid = lax.axis_index("s")
        wid = cid * NS + sid
        pltpu.sync_copy(z_hbm.at[pl.ds(sid * RPT, RPT)],
                        acc.at[pl.ds(sid * RPT, RPT)])
        pltpu.sync_copy(src_hbm.at[pl.ds(wid * NCH, NCH)], src_v)
        pltpu.sync_copy(dst_hbm.at[pl.ds(wid * NCH, NCH)], dst_v)
        plsc.subcore_barrier()

        def gstart(j, b):
            pltpu.async_copy(x_hbm.at[src_v.at[j]], rows_v.at[b], gsem[b])

        def gwait(j, b):
            pltpu.make_async_copy(x_hbm.at[src_v.at[j]], rows_v.at[b],
                                  gsem[b]).wait()

        def sstart(j, b):
            pltpu.async_copy(rows_v.at[b], acc.at[dst_v.at[j]], ssem[b],
                             add=True)

        def swait(j, b):
            pltpu.make_async_copy(rows_v.at[b], acc.at[dst_v.at[j]],
                                  ssem[b]).wait()

        for b in range(LB):
            gstart(b, b)
        for b in range(LB - GA):
            gwait(b, b)
            sstart(b, b)

        def body(t, carry):
            base = LB * t
            for b in range(LB):
                j = base + b
                swait(j - LB, b)
                gstart(j, b)
                bg = (b - GA) % LB
                gwait(j - GA, bg)
                sstart(j - GA, bg)
            return carry

        lax.fori_loop(1, NCH // LB, body, 0)
        for j in range(LB * (NCH // LB), NCH):
            b = j % LB
            swait(j - LB, b)
            gstart(j, b)
            gwait(j - GA, (j - GA) % LB)
            sstart(j - GA, (j - GA) % LB)
        for j in range(NCH, NCH + GA):
            swait(j - LB, (j - LB) % LB)
            gwait(j - GA, (j - GA) % LB)
            sstart(j - GA, (j - GA) % LB)
        for j in range(NCH + GA, NCH + LB):
            swait(j - LB, (j - LB) % LB)
        plsc.subcore_barrier()
        pltpu.sync_copy(acc.at[pl.ds(sid * RPT, RPT)],
                        out_hbm.at[pl.ds(cid * N + sid * RPT, RPT)])

    return k(x, src_r, dst_r, zeros)


def _elu(y):
    return jnp.where(y > 0, y, jnp.exp(jnp.minimum(y, 0.0)) - 1.0)


def _bn(y, s, g, be):
    m = s[0:1, :] * (1.0 / N)
    v = s[1:2, :] * (1.0 / N) - m * m
    return (y - m) * lax.rsqrt(v + EPS) * g + be


def _moments(y):
    s = jnp.concatenate(
        [jnp.sum(y, axis=0, keepdims=True),
         jnp.sum(y * y, axis=0, keepdims=True)], axis=0)
    return jnp.concatenate([s, jnp.zeros((6, y.shape[1]), jnp.float32)],
                           axis=0)


def _acc_add(ref, val, first):
    @pl.when(first)
    def _():
        ref[...] = val

    @pl.when(jnp.logical_not(first))
    def _():
        ref[...] = ref[...] + val


def _acc_max(ref, val, first):
    @pl.when(first)
    def _():
        ref[...] = val

    @pl.when(jnp.logical_not(first))
    def _():
        ref[...] = jnp.maximum(ref[...], val)


def _segmax(z, mask):
    cols = []
    for c in range(DT):
        mc = jnp.max(jnp.where(mask, z[:, c:c + 1], -jnp.inf), axis=0,
                     keepdims=True)
        cols.append(mc)
    return jnp.concatenate(cols, axis=0)


def _full(shape):
    return pl.BlockSpec(shape, lambda s: (0, 0))


def _blkA(w):
    return pl.BlockSpec((BR, w), lambda s: (jnp.minimum(s, NB - 1), 0))


def _blkB(w):
    return pl.BlockSpec((BR, w), lambda s: (jnp.maximum(s - NB, 0), 0))


def _mask_of(batch_blk):
    gids = lax.broadcasted_iota(jnp.int32, (1, G), 1)
    return batch_blk == gids


def _k1(x_ref, aa_ref, ab_ref, W0_ref, b0_ref, Wc_ref, bc_ref,
        g0_ref, be0_ref, gc_ref, bec_ref, W1_ref, b1_ref, batch_ref,
        z0_ref, x1_ref, yz_ref, sz_ref, m0_ref,
        y0s, y1s, s0a, s1a, sza, m0a):
    s = pl.program_id(0)

    @pl.when(s < NB)
    def _():
        x = x_ref[...]
        y0 = jnp.dot(x, W0_ref[...],
                     preferred_element_type=jnp.float32) + b0_ref[...]
        xin = x + aa_ref[...] + ab_ref[...]
        y1 = jnp.dot(xin, Wc_ref[...],
                     preferred_element_type=jnp.float32) + bc_ref[...]
        y0s[pl.ds(s * BR, BR), :] = y0
        y1s[pl.ds(s * BR, BR), :] = y1
        _acc_add(s0a, _moments(y0), s == 0)
        _acc_add(s1a, _moments(y1), s == 0)

    @pl.when(s >= NB)
    def _():
        i = s - NB
        y0 = y0s[pl.ds(i * BR, BR), :]
        y1 = y1s[pl.ds(i * BR, BR), :]
        z0 = _elu(_bn(y0, s0a[...], g0_ref[...], be0_ref[...]))
        x1 = _elu(_bn(y1, s1a[...], gc_ref[...], bec_ref[...]))
        z0_ref[...] = z0
        x1_ref[...] = x1
        yz = jnp.dot(x1, W1_ref[...],
                     preferred_element_type=jnp.float32) + b1_ref[...]
        yz_ref[...] = yz
        _acc_add(sza, _moments(yz), i == 0)
        _acc_max(m0a, _segmax(z0, _mask_of(batch_ref[...])), i == 0)

    @pl.when(s == 2 * NB - 1)
    def _():
        sz_ref[...] = sza[...]
        m0_ref[...] = m0a[...]


def _layer1(x, aggp, W0, b0, Wc, bc, g0, be0, gc, bec, W1, b1, batch2):
    return pl.pallas_call(
        _k1,
        grid=(2 * NB,),
        in_specs=[
            _blkA(D),
            pl.BlockSpec((BR, D), lambda s: (jnp.minimum(s, NB - 1), 0)),
            pl.BlockSpec((BR, D), lambda s: (jnp.minimum(s, NB - 1) + NB, 0)),
            _full((D, DT)), _full((1, DT)), _full((D, H)), _full((1, H)),
            _full((1, DT)), _full((1, DT)), _full((1, H)), _full((1, H)),
            _full((H, DT)), _full((1, DT)),
            pl.BlockSpec((BR, 1), lambda s: (jnp.maximum(s - NB, 0), 0)),
        ],
        out_specs=[_blkB(DT), _blkB(H), _blkB(DT), _full((8, DT)),
                   _full((DT, G))],
        out_shape=[
            jax.ShapeDtypeStruct((N, DT), jnp.float32),
            jax.ShapeDtypeStruct((N, H), jnp.float32),
            jax.ShapeDtypeStruct((N, DT), jnp.float32),
            jax.ShapeDtypeStruct((8, DT), jnp.float32),
            jax.ShapeDtypeStruct((DT, G), jnp.float32),
        ],
        scratch_shapes=[
            pltpu.VMEM((N, DT), jnp.float32),
            pltpu.VMEM((N, H), jnp.float32),
            pltpu.VMEM((8, DT), jnp.float32),
            pltpu.VMEM((8, H), jnp.float32),
            pltpu.VMEM((8, DT), jnp.float32),
            pltpu.VMEM((DT, G), jnp.float32),
        ],
    )(x, aggp, aggp, W0, b0, Wc, bc, g0, be0, gc, bec, W1, b1, batch2)


def _k2(yz_ref, sz_ref, g1_ref, be1_ref, batch_ref, z0_ref,
        zp_ref, m1_ref, z1s, esa, m1a):
    s = pl.program_id(0)

    @pl.when(s < NB)
    def _():
        z1 = _elu(_bn(yz_ref[...], sz_ref[...], g1_ref[...], be1_ref[...]))
        z1s[pl.ds(s * BR, BR), :] = z1
        e = jnp.sum(jnp.exp(z1), axis=0, keepdims=True)
        e = jnp.concatenate([e, jnp.zeros((7, DT), jnp.float32)], axis=0)
        _acc_add(esa, e, s == 0)
        _acc_max(m1a, _segmax(z1, _mask_of(batch_ref[...])), s == 0)

    @pl.when(s >= NB)
    def _():
        i = s - NB
        z1 = z1s[pl.ds(i * BR, BR), :]
        zp_ref[...] = z0_ref[...] + (jnp.exp(z1) / esa[0:1, :]) * z1

    @pl.when(s == 2 * NB - 1)
    def _():
        m1_ref[...] = m1a[...]


def _layer2(yz, sz, g1, be1, batch2, z0):
    return pl.pallas_call(
        _k2,
        grid=(2 * NB,),
        in_specs=[
            _blkA(DT), _full((8, DT)), _full((1, DT)), _full((1, DT)),
            pl.BlockSpec((BR, 1), lambda s: (jnp.minimum(s, NB - 1), 0)),
            _blkB(DT),
        ],
        out_specs=[_blkB(DT), _full((DT, G))],
        out_shape=[
            jax.ShapeDtypeStruct((N, DT), jnp.float32),
            jax.ShapeDtypeStruct((DT, G), jnp.float32),
        ],
        scratch_shapes=[
            pltpu.VMEM((N, DT), jnp.float32),
            pltpu.VMEM((8, DT), jnp.float32),
            pltpu.VMEM((DT, G), jnp.float32),
        ],
    )(yz, sz, g1, be1, batch2, z0)


def _k3(x1_ref, aa_ref, ab_ref, Wc_ref, bc_ref, gc_ref, bec_ref,
        W2_ref, b2_ref,
        x2_ref, yz_ref, sz_ref, y2s, s2a, sza):
    s = pl.program_id(0)

    @pl.when(s < NB)
    def _():
        xin = x1_ref[...] + aa_ref[...] + ab_ref[...]
        y2 = jnp.dot(xin, Wc_ref[...],
                     preferred_element_type=jnp.float32) + bc_ref[...]
        y2s[pl.ds(s * BR, BR), :] = y2
        _acc_add(s2a, _moments(y2), s == 0)

    @pl.when(s >= NB)
    def _():
        i = s - NB
        y2 = y2s[pl.ds(i * BR, BR), :]
        x2 = _elu(_bn(y2, s2a[...], gc_ref[...], bec_ref[...]))
        x2_ref[...] = x2
        yz = jnp.dot(x2, W2_ref[...],
                     preferred_element_type=jnp.float32) + b2_ref[...]
        yz_ref[...] = yz
        _acc_add(sza, _moments(yz), i == 0)

    @pl.when(s == 2 * NB - 1)
    def _():
        sz_ref[...] = sza[...]


def _layer3(x1, aggp, Wc, bc, gc, bec, W2, b2):
    return pl.pallas_call(
        _k3,
        grid=(2 * NB,),
        in_specs=[
            _blkA(D),
            pl.BlockSpec((BR, D), lambda s: (jnp.minimum(s, NB - 1), 0)),
            pl.BlockSpec((BR, D), lambda s: (jnp.minimum(s, NB - 1) + NB, 0)),
            _full((D, H2)), _full((1, H2)), _full((1, H2)), _full((1, H2)),
            _full((H2, DT)), _full((1, DT)),
        ],
        out_specs=[_blkB(H2), _blkB(DT), _full((8, DT))],
        out_shape=[
            jax.ShapeDtypeStruct((N, H2), jnp.float32),
            jax.ShapeDtypeStruct((N, DT), jnp.float32),
            jax.ShapeDtypeStruct((8, DT), jnp.float32),
        ],
        scratch_shapes=[
            pltpu.VMEM((N, H2), jnp.float32),
            pltpu.VMEM((8, H2), jnp.float32),
            pltpu.VMEM((8, DT), jnp.float32),
        ],
    )(x1, aggp, aggp, Wc, bc, gc, bec, W2, b2)


def _k4(yz_ref, sz_ref, g2_ref, be2_ref, batch_ref, zp_ref, m0_ref, m1_ref,
        z_ref, ot_ref, z2s, esa, m2a):
    s = pl.program_id(0)

    @pl.when(s < NB)
    def _():
        z2 = _elu(_bn(yz_ref[...], sz_ref[...], g2_ref[...], be2_ref[...]))
        z2s[pl.ds(s * BR, BR), :] = z2
        e = jnp.sum(jnp.exp(z2), axis=0, keepdims=True)
        e = jnp.concatenate([e, jnp.zeros((7, DT), jnp.float32)], axis=0)
        _acc_add(esa, e, s == 0)
        _acc_max(m2a, _segmax(z2, _mask_of(batch_ref[...])), s == 0)

    @pl.when(s >= NB)
    def _():
        i = s - NB
        z2 = z2s[pl.ds(i * BR, BR), :]
        z_ref[...] = zp_ref[...] + (jnp.exp(z2) / esa[0:1, :]) * z2

    @pl.when(s == 2 * NB - 1)
    def _():
        ot_ref[...] = m0_ref[...] + m1_ref[...] + m2a[...]


def _layer4(yz2, sz2, g2, be2, batch2, zp, m0, m1):
    return pl.pallas_call(
        _k4,
        grid=(2 * NB,),
        in_specs=[
            _blkA(DT), _full((8, DT)), _full((1, DT)), _full((1, DT)),
            pl.BlockSpec((BR, 1), lambda s: (jnp.minimum(s, NB - 1), 0)),
            _blkB(DT), _full((DT, G)), _full((DT, G)),
        ],
        out_specs=[_blkB(DT), _full((DT, G))],
        out_shape=[
            jax.ShapeDtypeStruct((N, DT), jnp.float32),
            jax.ShapeDtypeStruct((DT, G), jnp.float32),
        ],
        scratch_shapes=[
            pltpu.VMEM((N, DT), jnp.float32),
            pltpu.VMEM((8, DT), jnp.float32),
            pltpu.VMEM((DT, G), jnp.float32),
        ],
    )(yz2, sz2, g2, be2, batch2, zp, m0, m1)


def kernel(x, edge_index, batch, W0, b0, g0, be0, Wc1, bc1, gc1, bec1,
           W1, b1, g1, be1, Wc2, bc2, gc2, bec2, W2, b2, g2, be2):
    src_r = edge_index[0].reshape(NW * NCH, CH)
    dst_r = edge_index[1].reshape(NW * NCH, CH)
    zeros = jnp.zeros((N, D), jnp.float32)
    batch2 = batch.reshape(N, 1)
    r2 = lambda a: a.reshape(1, -1)

    agg1p = _sc_segment_sum(x, src_r, dst_r, zeros)
    z0, x1, yz1, sz1, m0 = _layer1(x, agg1p, W0, r2(b0), Wc1, r2(bc1),
                                   r2(g0), r2(be0), r2(gc1), r2(bec1),
                                   W1, r2(b1), batch2)
    agg2p = _sc_segment_sum(x1, src_r, dst_r, zeros)
    Zp, m1 = _layer2(yz1, sz1, r2(g1), r2(be1), batch2, z0)
    x2, yz2, sz2 = _layer3(x1, agg2p, Wc2, r2(bc2), r2(gc2), r2(bec2),
                           W2, r2(b2))
    Z, out_t = _layer4(yz2, sz2, r2(g2), r2(be2), batch2, Zp, m0, m1)
    return out_t.T, Z, x2


# speedup vs baseline: 1.0848x; 1.0280x over previous
import functools

import jax
import jax.numpy as jnp
from jax import lax
from jax.experimental import pallas as pl
from jax.experimental.pallas import tpu as pltpu
from jax.experimental.pallas import tpu_sc as plsc

N = 10000
E = 320000
D = 128
H = 128
H2 = 64
DT = 16
G = 128

NC = 2
NS = 16
NW = NC * NS
EPW = E // NW
CH = 40
NCH = EPW // CH
RPT = N // NS
LB = 6
GA = 4

BR = 2000
NB = N // BR

EPS = 1e-5


def _sc_segment_sum(x, src_r, dst_r, zeros):
    mesh = plsc.VectorSubcoreMesh(core_axis_name="c", subcore_axis_name="s")

    @functools.partial(
        pl.kernel,
        out_type=jax.ShapeDtypeStruct((NC * N, D), jnp.float32),
        mesh=mesh,
        compiler_params=pltpu.CompilerParams(use_tc_tiling_on_sc=False),
        scratch_types=[
            pltpu.VMEM((NCH, CH), jnp.int32),
            pltpu.VMEM((NCH, CH), jnp.int32),
            pltpu.VMEM((LB, CH, D), jnp.float32),
            pltpu.VMEM_SHARED((N, D), jnp.float32),
        ] + [pltpu.SemaphoreType.DMA] * (2 * LB),
    )
    def k(x_hbm, src_hbm, dst_hbm, z_hbm, out_hbm, src_v, dst_v, rows_v, acc,
          *sems):
        gsem = sems[:LB]
        ssem = sems[LB:]
        cid = lax.axis_index("c")
        sid = lax.axis_index("s")
        wid = cid * NS + sid
        pltpu.sync_copy(z_hbm.at[pl.ds(sid * RPT, RPT)],
                        acc.at[pl.ds(sid * RPT, RPT)])
        pltpu.sync_copy(src_hbm.at[pl.ds(wid * NCH, NCH)], src_v)
        pltpu.sync_copy(dst_hbm.at[pl.ds(wid * NCH, NCH)], dst_v)
        plsc.subcore_barrier()

        def gstart(j, b):
            pltpu.async_copy(x_hbm.at[src_v.at[j]], rows_v.at[b], gsem[b])

        def gwait(j, b):
            pltpu.make_async_copy(x_hbm.at[src_v.at[j]], rows_v.at[b],
                                  gsem[b]).wait()

        def sstart(j, b):
            pltpu.async_copy(rows_v.at[b], acc.at[dst_v.at[j]], ssem[b],
                             add=True)

        def swait(j, b):
            pltpu.make_async_copy(rows_v.at[b], acc.at[dst_v.at[j]],
                                  ssem[b]).wait()

        for b in range(LB):
            gstart(b, b)
        for b in range(LB - GA):
            gwait(b, b)
            sstart(b, b)

        def body(t, carry):
            base = LB * t
            for b in range(LB):
                j = base + b
                swait(j - LB, b)
                gstart(j, b)
                bg = (b - GA) % LB
                gwait(j - GA, bg)
                sstart(j - GA, bg)
            return carry

        lax.fori_loop(1, NCH // LB, body, 0)
        for j in range(LB * (NCH // LB), NCH):
            b = j % LB
            swait(j - LB, b)
            gstart(j, b)
            gwait(j - GA, (j - GA) % LB)
            sstart(j - GA, (j - GA) % LB)
        for j in range(NCH, NCH + GA):
            swait(j - LB, (j - LB) % LB)
            gwait(j - GA, (j - GA) % LB)
            sstart(j - GA, (j - GA) % LB)
        for j in range(NCH + GA, NCH + LB):
            swait(j - LB, (j - LB) % LB)
        plsc.subcore_barrier()
        pltpu.sync_copy(acc.at[pl.ds(sid * RPT, RPT)],
                        out_hbm.at[pl.ds(cid * N + sid * RPT, RPT)])

    return k(x, src_r, dst_r, zeros)


def _elu(y):
    return jnp.where(y > 0, y, jnp.exp(jnp.minimum(y, 0.0)) - 1.0)


def _bn(y, s, g, be):
    m = s[0:1, :] * (1.0 / N)
    v = s[1:2, :] * (1.0 / N) - m * m
    return (y - m) * lax.rsqrt(v + EPS) * g + be


def _moments(y):
    s = jnp.concatenate(
        [jnp.sum(y, axis=0, keepdims=True),
         jnp.sum(y * y, axis=0, keepdims=True)], axis=0)
    return jnp.concatenate([s, jnp.zeros((6, y.shape[1]), jnp.float32)],
                           axis=0)


def _acc_add(ref, val, first):
    @pl.when(first)
    def _():
        ref[...] = val

    @pl.when(jnp.logical_not(first))
    def _():
        ref[...] = ref[...] + val


def _acc_max(ref, val, first):
    @pl.when(first)
    def _():
        ref[...] = val

    @pl.when(jnp.logical_not(first))
    def _():
        ref[...] = jnp.maximum(ref[...], val)


def _segmax(z, mask):
    cols = []
    for c in range(DT):
        mc = jnp.max(jnp.where(mask, z[:, c:c + 1], -jnp.inf), axis=0,
                     keepdims=True)
        cols.append(mc)
    return jnp.concatenate(cols, axis=0)


def _full(shape):
    return pl.BlockSpec(shape, lambda s: (0, 0))


def _blkA(w):
    return pl.BlockSpec((BR, w), lambda s: (jnp.minimum(s, NB - 1), 0))


def _blkB(w):
    return pl.BlockSpec((BR, w), lambda s: (jnp.maximum(s - NB, 0), 0))


def _mask_of(batch_blk):
    gids = lax.broadcasted_iota(jnp.int32, (1, G), 1)
    return batch_blk == gids


def _k1(x_ref, aa_ref, ab_ref, W0_ref, b0_ref, Wc_ref, bc_ref,
        g0_ref, be0_ref, gc_ref, bec_ref, W1_ref, b1_ref, batch_ref,
        z0_ref, x1_ref, yz_ref, sz_ref, m0_ref,
        y0s, y1s, s0a, s1a, sza, m0a):
    s = pl.program_id(0)

    @pl.when(s < NB)
    def _():
        x = x_ref[...]
        y0 = jnp.dot(x, W0_ref[...],
                     preferred_element_type=jnp.float32) + b0_ref[...]
        xin = x + aa_ref[...] + ab_ref[...]
        y1 = jnp.dot(xin, Wc_ref[...],
                     preferred_element_type=jnp.float32) + bc_ref[...]
        y0s[pl.ds(s * BR, BR), :] = y0
        y1s[pl.ds(s * BR, BR), :] = y1
        _acc_add(s0a, _moments(y0), s == 0)
        _acc_add(s1a, _moments(y1), s == 0)

    @pl.when(s >= NB)
    def _():
        i = s - NB
        y0 = y0s[pl.ds(i * BR, BR), :]
        y1 = y1s[pl.ds(i * BR, BR), :]
        z0 = _elu(_bn(y0, s0a[...], g0_ref[...], be0_ref[...]))
        x1 = _elu(_bn(y1, s1a[...], gc_ref[...], bec_ref[...]))
        z0_ref[...] = z0
        x1_ref[...] = x1
        yz = jnp.dot(x1, W1_ref[...],
                     preferred_element_type=jnp.float32) + b1_ref[...]
        yz_ref[...] = yz
        _acc_add(sza, _moments(yz), i == 0)
        _acc_max(m0a, _segmax(z0, _mask_of(batch_ref[...])), i == 0)

    @pl.when(s == 2 * NB - 1)
    def _():
        sz_ref[...] = sza[...]
        m0_ref[...] = m0a[...]


def _layer1(x, aggp, W0, b0, Wc, bc, g0, be0, gc, bec, W1, b1, batch2):
    return pl.pallas_call(
        _k1,
        grid=(2 * NB,),
        in_specs=[
            _blkA(D),
            pl.BlockSpec((BR, D), lambda s: (jnp.minimum(s, NB - 1), 0)),
            pl.BlockSpec((BR, D), lambda s: (jnp.minimum(s, NB - 1) + NB, 0)),
            _full((D, DT)), _full((1, DT)), _full((D, H)), _full((1, H)),
            _full((1, DT)), _full((1, DT)), _full((1, H)), _full((1, H)),
            _full((H, DT)), _full((1, DT)),
            pl.BlockSpec((BR, 1), lambda s: (jnp.maximum(s - NB, 0), 0)),
        ],
        out_specs=[_blkB(DT), _blkB(H), _blkB(DT), _full((8, DT)),
                   _full((DT, G))],
        out_shape=[
            jax.ShapeDtypeStruct((N, DT), jnp.float32),
            jax.ShapeDtypeStruct((N, H), jnp.float32),
            jax.ShapeDtypeStruct((N, DT), jnp.float32),
            jax.ShapeDtypeStruct((8, DT), jnp.float32),
            jax.ShapeDtypeStruct((DT, G), jnp.float32),
        ],
        scratch_shapes=[
            pltpu.VMEM((N, DT), jnp.float32),
            pltpu.VMEM((N, H), jnp.float32),
            pltpu.VMEM((8, DT), jnp.float32),
            pltpu.VMEM((8, H), jnp.float32),
            pltpu.VMEM((8, DT), jnp.float32),
            pltpu.VMEM((DT, G), jnp.float32),
        ],
    )(x, aggp, aggp, W0, b0, Wc, bc, g0, be0, gc, bec, W1, b1, batch2)


def _k2(yz_ref, sz_ref, g1_ref, be1_ref, batch_ref, z0_ref,
        zp_ref, m1_ref, z1s, esa, m1a):
    s = pl.program_id(0)

    @pl.when(s < NB)
    def _():
        z1 = _elu(_bn(yz_ref[...], sz_ref[...], g1_ref[...], be1_ref[...]))
        z1s[pl.ds(s * BR, BR), :] = z1
        e = jnp.sum(jnp.exp(z1), axis=0, keepdims=True)
        e = jnp.concatenate([e, jnp.zeros((7, DT), jnp.float32)], axis=0)
        _acc_add(esa, e, s == 0)
        _acc_max(m1a, _segmax(z1, _mask_of(batch_ref[...])), s == 0)

    @pl.when(s >= NB)
    def _():
        i = s - NB
        z1 = z1s[pl.ds(i * BR, BR), :]
        zp_ref[...] = z0_ref[...] + (jnp.exp(z1) / esa[0:1, :]) * z1

    @pl.when(s == 2 * NB - 1)
    def _():
        m1_ref[...] = m1a[...]


def _layer2(yz, sz, g1, be1, batch2, z0):
    return pl.pallas_call(
        _k2,
        grid=(2 * NB,),
        in_specs=[
            _blkA(DT), _full((8, DT)), _full((1, DT)), _full((1, DT)),
            pl.BlockSpec((BR, 1), lambda s: (jnp.minimum(s, NB - 1), 0)),
            _blkB(DT),
        ],
        out_specs=[_blkB(DT), _full((DT, G))],
        out_shape=[
            jax.ShapeDtypeStruct((N, DT), jnp.float32),
            jax.ShapeDtypeStruct((DT, G), jnp.float32),
        ],
        scratch_shapes=[
            pltpu.VMEM((N, DT), jnp.float32),
            pltpu.VMEM((8, DT), jnp.float32),
            pltpu.VMEM((DT, G), jnp.float32),
        ],
    )(yz, sz, g1, be1, batch2, z0)


def _k3(x1_ref, aa_ref, ab_ref, Wc_ref, bc_ref, gc_ref, bec_ref,
        W2_ref, b2_ref,
        x2_ref, yz_ref, sz_ref, y2s, s2a, sza):
    s = pl.program_id(0)

    @pl.when(s < NB)
    def _():
        xin = x1_ref[...] + aa_ref[...] + ab_ref[...]
        y2 = jnp.dot(xin, Wc_ref[...],
                     preferred_element_type=jnp.float32) + bc_ref[...]
        y2s[pl.ds(s * BR, BR), :] = y2
        _acc_add(s2a, _moments(y2), s == 0)

    @pl.when(s >= NB)
    def _():
        i = s - NB
        y2 = y2s[pl.ds(i * BR, BR), :]
        x2 = _elu(_bn(y2, s2a[...], gc_ref[...], bec_ref[...]))
        x2_ref[...] = x2
        yz = jnp.dot(x2, W2_ref[...],
                     preferred_element_type=jnp.float32) + b2_ref[...]
        yz_ref[...] = yz
        _acc_add(sza, _moments(yz), i == 0)

    @pl.when(s == 2 * NB - 1)
    def _():
        sz_ref[...] = sza[...]


def _layer3(x1, aggp, Wc, bc, gc, bec, W2, b2):
    return pl.pallas_call(
        _k3,
        grid=(2 * NB,),
        in_specs=[
            _blkA(D),
            pl.BlockSpec((BR, D), lambda s: (jnp.minimum(s, NB - 1), 0)),
            pl.BlockSpec((BR, D), lambda s: (jnp.minimum(s, NB - 1) + NB, 0)),
            _full((D, H2)), _full((1, H2)), _full((1, H2)), _full((1, H2)),
            _full((H2, DT)), _full((1, DT)),
        ],
        out_specs=[_blkB(H2), _blkB(DT), _full((8, DT))],
        out_shape=[
            jax.ShapeDtypeStruct((N, H2), jnp.float32),
            jax.ShapeDtypeStruct((N, DT), jnp.float32),
            jax.ShapeDtypeStruct((8, DT), jnp.float32),
        ],
        scratch_shapes=[
            pltpu.VMEM((N, H2), jnp.float32),
            pltpu.VMEM((8, H2), jnp.float32),
            pltpu.VMEM((8, DT), jnp.float32),
        ],
    )(x1, aggp, aggp, Wc, bc, gc, bec, W2, b2)


def _k4(yz_ref, sz_ref, g2_ref, be2_ref, batch_ref, zp_ref, m0_ref, m1_ref,
        z_ref, ot_ref, z2s, esa, m2a):
    s = pl.program_id(0)

    @pl.when(s < NB)
    def _():
        z2 = _elu(_bn(yz_ref[...], sz_ref[...], g2_ref[...], be2_ref[...]))
        z2s[pl.ds(s * BR, BR), :] = z2
        e = jnp.sum(jnp.exp(z2), axis=0, keepdims=True)
        e = jnp.concatenate([e, jnp.zeros((7, DT), jnp.float32)], axis=0)
        _acc_add(esa, e, s == 0)
        _acc_max(m2a, _segmax(z2, _mask_of(batch_ref[...])), s == 0)

    @pl.when(s >= NB)
    def _():
        i = s - NB
        z2 = z2s[pl.ds(i * BR, BR), :]
        z_ref[...] = zp_ref[...] + (jnp.exp(z2) / esa[0:1, :]) * z2

    @pl.when(s == 2 * NB - 1)
    def _():
        ot_ref[...] = m0_ref[...] + m1_ref[...] + m2a[...]


def _layer4(yz2, sz2, g2, be2, batch2, zp, m0, m1):
    return pl.pallas_call(
        _k4,
        grid=(2 * NB,),
        in_specs=[
            _blkA(DT), _full((8, DT)), _full((1, DT)), _full((1, DT)),
            pl.BlockSpec((BR, 1), lambda s: (jnp.minimum(s, NB - 1), 0)),
            _blkB(DT), _full((DT, G)), _full((DT, G)),
        ],
        out_specs=[_blkB(DT), _full((DT, G))],
        out_shape=[
            jax.ShapeDtypeStruct((N, DT), jnp.float32),
            jax.ShapeDtypeStruct((DT, G), jnp.float32),
        ],
        scratch_shapes=[
            pltpu.VMEM((N, DT), jnp.float32),
            pltpu.VMEM((8, DT), jnp.float32),
            pltpu.VMEM((DT, G), jnp.float32),
        ],
    )(yz2, sz2, g2, be2, batch2, zp, m0, m1)


def kernel(x, edge_index, batch, W0, b0, g0, be0, Wc1, bc1, gc1, bec1,
           W1, b1, g1, be1, Wc2, bc2, gc2, bec2, W2, b2, g2, be2):
    src_r = edge_index[0].reshape(NW * NCH, CH)
    dst_r = edge_index[1].reshape(NW * NCH, CH)
    zeros = jnp.zeros((N, D), jnp.float32)
    batch2 = batch.reshape(N, 1)
    r2 = lambda a: a.reshape(1, -1)

    agg1p = _sc_segment_sum(x, src_r, dst_r, zeros)
    z0, x1, yz1, sz1, m0 = _layer1(x, agg1p, W0, r2(b0), Wc1, r2(bc1),
                                   r2(g0), r2(be0), r2(gc1), r2(bec1),
                                   W1, r2(b1), batch2)
    agg2p = _sc_segment_sum(x1, src_r, dst_r, zeros)
    Zp, m1 = _layer2(yz1, sz1, r2(g1), r2(be1), batch2, z0)
    x2, yz2, sz2 = _layer3(x1, agg2p, Wc2, r2(bc2), r2(gc2), r2(bec2),
                           W2, r2(b2))
    Z, out_t = _layer4(yz2, sz2, r2(g2), r2(be2), batch2, Zp, m0, m1)
    return out_t.T, Z, x2


# speedup vs baseline: 1.0945x; 1.0089x over previous
import functools

import jax
import jax.numpy as jnp
from jax import lax
from jax.experimental import pallas as pl
from jax.experimental.pallas import tpu as pltpu
from jax.experimental.pallas import tpu_sc as plsc

N = 10000
E = 320000
D = 128
H = 128
H2 = 64
DT = 16
G = 128

NC = 2
NS = 16
NW = NC * NS
EPW = E // NW
CH = 40
NCH = EPW // CH
RPT = N // NS
LB = 6
GA = 5

BR = 2000
NB = N // BR

EPS = 1e-5


def _sc_segment_sum(x, src_r, dst_r, zeros):
    mesh = plsc.VectorSubcoreMesh(core_axis_name="c", subcore_axis_name="s")

    @functools.partial(
        pl.kernel,
        out_type=jax.ShapeDtypeStruct((NC * N, D), jnp.float32),
        mesh=mesh,
        compiler_params=pltpu.CompilerParams(use_tc_tiling_on_sc=False),
        scratch_types=[
            pltpu.VMEM((NCH, CH), jnp.int32),
            pltpu.VMEM((NCH, CH), jnp.int32),
            pltpu.VMEM((LB, CH, D), jnp.float32),
            pltpu.VMEM_SHARED((N, D), jnp.float32),
        ] + [pltpu.SemaphoreType.DMA] * (2 * LB),
    )
    def k(x_hbm, src_hbm, dst_hbm, z_hbm, out_hbm, src_v, dst_v, rows_v, acc,
          *sems):
        gsem = sems[:LB]
        ssem = sems[LB:]
        cid = lax.axis_index("c")
        sid = lax.axis_index("s")
        wid = cid * NS + sid
        pltpu.sync_copy(z_hbm.at[pl.ds(sid * RPT, RPT)],
                        acc.at[pl.ds(sid * RPT, RPT)])
        pltpu.sync_copy(src_hbm.at[pl.ds(wid * NCH, NCH)], src_v)
        pltpu.sync_copy(dst_hbm.at[pl.ds(wid * NCH, NCH)], dst_v)
        plsc.subcore_barrier()

        def gstart(j, b):
            pltpu.async_copy(x_hbm.at[src_v.at[j]], rows_v.at[b], gsem[b])

        def gwait(j, b):
            pltpu.make_async_copy(x_hbm.at[src_v.at[j]], rows_v.at[b],
                                  gsem[b]).wait()

        def sstart(j, b):
            pltpu.async_copy(rows_v.at[b], acc.at[dst_v.at[j]], ssem[b],
                             add=True)

        def swait(j, b):
            pltpu.make_async_copy(rows_v.at[b], acc.at[dst_v.at[j]],
                                  ssem[b]).wait()

        for b in range(LB):
            gstart(b, b)
        for b in range(LB - GA):
            gwait(b, b)
            sstart(b, b)

        def body(t, carry):
            base = LB * t
            for b in range(LB):
                j = base + b
                swait(j - LB, b)
                gstart(j, b)
                bg = (b - GA) % LB
                gwait(j - GA, bg)
                sstart(j - GA, bg)
            return carry

        lax.fori_loop(1, NCH // LB, body, 0)
        for j in range(LB * (NCH // LB), NCH):
            b = j % LB
            swait(j - LB, b)
            gstart(j, b)
            gwait(j - GA, (j - GA) % LB)
            sstart(j - GA, (j - GA) % LB)
        for j in range(NCH, NCH + GA):
            swait(j - LB, (j - LB) % LB)
            gwait(j - GA, (j - GA) % LB)
            sstart(j - GA, (j - GA) % LB)
        for j in range(NCH + GA, NCH + LB):
            swait(j - LB, (j - LB) % LB)
        plsc.subcore_barrier()
        pltpu.sync_copy(acc.at[pl.ds(sid * RPT, RPT)],
                        out_hbm.at[pl.ds(cid * N + sid * RPT, RPT)])

    return k(x, src_r, dst_r, zeros)


def _elu(y):
    return jnp.where(y > 0, y, jnp.exp(jnp.minimum(y, 0.0)) - 1.0)


def _bn(y, s, g, be):
    m = s[0:1, :] * (1.0 / N)
    v = s[1:2, :] * (1.0 / N) - m * m
    return (y - m) * lax.rsqrt(v + EPS) * g + be


def _moments(y):
    s = jnp.concatenate(
        [jnp.sum(y, axis=0, keepdims=True),
         jnp.sum(y * y, axis=0, keepdims=True)], axis=0)
    return jnp.concatenate([s, jnp.zeros((6, y.shape[1]), jnp.float32)],
                           axis=0)


def _acc_add(ref, val, first):
    @pl.when(first)
    def _():
        ref[...] = val

    @pl.when(jnp.logical_not(first))
    def _():
        ref[...] = ref[...] + val


def _acc_max(ref, val, first):
    @pl.when(first)
    def _():
        ref[...] = val

    @pl.when(jnp.logical_not(first))
    def _():
        ref[...] = jnp.maximum(ref[...], val)


def _segmax(z, mask):
    cols = []
    for c in range(DT):
        mc = jnp.max(jnp.where(mask, z[:, c:c + 1], -jnp.inf), axis=0,
                     keepdims=True)
        cols.append(mc)
    return jnp.concatenate(cols, axis=0)


def _full(shape):
    return pl.BlockSpec(shape, lambda s: (0, 0))


def _blkA(w):
    return pl.BlockSpec((BR, w), lambda s: (jnp.minimum(s, NB - 1), 0))


def _blkB(w):
    return pl.BlockSpec((BR, w), lambda s: (jnp.maximum(s - NB, 0), 0))


def _mask_of(batch_blk):
    gids = lax.broadcasted_iota(jnp.int32, (1, G), 1)
    return batch_blk == gids


def _k1(x_ref, aa_ref, ab_ref, W0_ref, b0_ref, Wc_ref, bc_ref,
        g0_ref, be0_ref, gc_ref, bec_ref, W1_ref, b1_ref, batch_ref,
        z0_ref, x1_ref, yz_ref, sz_ref, m0_ref,
        y0s, y1s, s0a, s1a, sza, m0a):
    s = pl.program_id(0)

    @pl.when(s < NB)
    def _():
        x = x_ref[...]
        y0 = jnp.dot(x, W0_ref[...],
                     preferred_element_type=jnp.float32) + b0_ref[...]
        xin = x + aa_ref[...] + ab_ref[...]
        y1 = jnp.dot(xin, Wc_ref[...],
                     preferred_element_type=jnp.float32) + bc_ref[...]
        y0s[pl.ds(s * BR, BR), :] = y0
        y1s[pl.ds(s * BR, BR), :] = y1
        _acc_add(s0a, _moments(y0), s == 0)
        _acc_add(s1a, _moments(y1), s == 0)

    @pl.when(s >= NB)
    def _():
        i = s - NB
        y0 = y0s[pl.ds(i * BR, BR), :]
        y1 = y1s[pl.ds(i * BR, BR), :]
        z0 = _elu(_bn(y0, s0a[...], g0_ref[...], be0_ref[...]))
        x1 = _elu(_bn(y1, s1a[...], gc_ref[...], bec_ref[...]))
        z0_ref[...] = z0
        x1_ref[...] = x1
        yz = jnp.dot(x1, W1_ref[...],
                     preferred_element_type=jnp.float32) + b1_ref[...]
        yz_ref[...] = yz
        _acc_add(sza, _moments(yz), i == 0)
        _acc_max(m0a, _segmax(z0, _mask_of(batch_ref[...])), i == 0)

    @pl.when(s == 2 * NB - 1)
    def _():
        sz_ref[...] = sza[...]
        m0_ref[...] = m0a[...]


def _layer1(x, aggp, W0, b0, Wc, bc, g0, be0, gc, bec, W1, b1, batch2):
    return pl.pallas_call(
        _k1,
        grid=(2 * NB,),
        in_specs=[
            _blkA(D),
            pl.BlockSpec((BR, D), lambda s: (jnp.minimum(s, NB - 1), 0)),
            pl.BlockSpec((BR, D), lambda s: (jnp.minimum(s, NB - 1) + NB, 0)),
            _full((D, DT)), _full((1, DT)), _full((D, H)), _full((1, H)),
            _full((1, DT)), _full((1, DT)), _full((1, H)), _full((1, H)),
            _full((H, DT)), _full((1, DT)),
            pl.BlockSpec((BR, 1), lambda s: (jnp.maximum(s - NB, 0), 0)),
        ],
        out_specs=[_blkB(DT), _blkB(H), _blkB(DT), _full((8, DT)),
                   _full((DT, G))],
        out_shape=[
            jax.ShapeDtypeStruct((N, DT), jnp.float32),
            jax.ShapeDtypeStruct((N, H), jnp.float32),
            jax.ShapeDtypeStruct((N, DT), jnp.float32),
            jax.ShapeDtypeStruct((8, DT), jnp.float32),
            jax.ShapeDtypeStruct((DT, G), jnp.float32),
        ],
        scratch_shapes=[
            pltpu.VMEM((N, DT), jnp.float32),
            pltpu.VMEM((N, H), jnp.float32),
            pltpu.VMEM((8, DT), jnp.float32),
            pltpu.VMEM((8, H), jnp.float32),
            pltpu.VMEM((8, DT), jnp.float32),
            pltpu.VMEM((DT, G), jnp.float32),
        ],
    )(x, aggp, aggp, W0, b0, Wc, bc, g0, be0, gc, bec, W1, b1, batch2)


def _k2(yz_ref, sz_ref, g1_ref, be1_ref, batch_ref, z0_ref,
        zp_ref, m1_ref, z1s, esa, m1a):
    s = pl.program_id(0)

    @pl.when(s < NB)
    def _():
        z1 = _elu(_bn(yz_ref[...], sz_ref[...], g1_ref[...], be1_ref[...]))
        z1s[pl.ds(s * BR, BR), :] = z1
        e = jnp.sum(jnp.exp(z1), axis=0, keepdims=True)
        e = jnp.concatenate([e, jnp.zeros((7, DT), jnp.float32)], axis=0)
        _acc_add(esa, e, s == 0)
        _acc_max(m1a, _segmax(z1, _mask_of(batch_ref[...])), s == 0)

    @pl.when(s >= NB)
    def _():
        i = s - NB
        z1 = z1s[pl.ds(i * BR, BR), :]
        zp_ref[...] = z0_ref[...] + (jnp.exp(z1) / esa[0:1, :]) * z1

    @pl.when(s == 2 * NB - 1)
    def _():
        m1_ref[...] = m1a[...]


def _layer2(yz, sz, g1, be1, batch2, z0):
    return pl.pallas_call(
        _k2,
        grid=(2 * NB,),
        in_specs=[
            _blkA(DT), _full((8, DT)), _full((1, DT)), _full((1, DT)),
            pl.BlockSpec((BR, 1), lambda s: (jnp.minimum(s, NB - 1), 0)),
            _blkB(DT),
        ],
        out_specs=[_blkB(DT), _full((DT, G))],
        out_shape=[
            jax.ShapeDtypeStruct((N, DT), jnp.float32),
            jax.ShapeDtypeStruct((DT, G), jnp.float32),
        ],
        scratch_shapes=[
            pltpu.VMEM((N, DT), jnp.float32),
            pltpu.VMEM((8, DT), jnp.float32),
            pltpu.VMEM((DT, G), jnp.float32),
        ],
    )(yz, sz, g1, be1, batch2, z0)


def _k3(x1_ref, aa_ref, ab_ref, Wc_ref, bc_ref, gc_ref, bec_ref,
        W2_ref, b2_ref,
        x2_ref, yz_ref, sz_ref, y2s, s2a, sza):
    s = pl.program_id(0)

    @pl.when(s < NB)
    def _():
        xin = x1_ref[...] + aa_ref[...] + ab_ref[...]
        y2 = jnp.dot(xin, Wc_ref[...],
                     preferred_element_type=jnp.float32) + bc_ref[...]
        y2s[pl.ds(s * BR, BR), :] = y2
        _acc_add(s2a, _moments(y2), s == 0)

    @pl.when(s >= NB)
    def _():
        i = s - NB
        y2 = y2s[pl.ds(i * BR, BR), :]
        x2 = _elu(_bn(y2, s2a[...], gc_ref[...], bec_ref[...]))
        x2_ref[...] = x2
        yz = jnp.dot(x2, W2_ref[...],
                     preferred_element_type=jnp.float32) + b2_ref[...]
        yz_ref[...] = yz
        _acc_add(sza, _moments(yz), i == 0)

    @pl.when(s == 2 * NB - 1)
    def _():
        sz_ref[...] = sza[...]


def _layer3(x1, aggp, Wc, bc, gc, bec, W2, b2):
    return pl.pallas_call(
        _k3,
        grid=(2 * NB,),
        in_specs=[
            _blkA(D),
            pl.BlockSpec((BR, D), lambda s: (jnp.minimum(s, NB - 1), 0)),
            pl.BlockSpec((BR, D), lambda s: (jnp.minimum(s, NB - 1) + NB, 0)),
            _full((D, H2)), _full((1, H2)), _full((1, H2)), _full((1, H2)),
            _full((H2, DT)), _full((1, DT)),
        ],
        out_specs=[_blkB(H2), _blkB(DT), _full((8, DT))],
        out_shape=[
            jax.ShapeDtypeStruct((N, H2), jnp.float32),
            jax.ShapeDtypeStruct((N, DT), jnp.float32),
            jax.ShapeDtypeStruct((8, DT), jnp.float32),
        ],
        scratch_shapes=[
            pltpu.VMEM((N, H2), jnp.float32),
            pltpu.VMEM((8, H2), jnp.float32),
            pltpu.VMEM((8, DT), jnp.float32),
        ],
    )(x1, aggp, aggp, Wc, bc, gc, bec, W2, b2)


def _k4(yz_ref, sz_ref, g2_ref, be2_ref, batch_ref, zp_ref, m0_ref, m1_ref,
        z_ref, ot_ref, z2s, esa, m2a):
    s = pl.program_id(0)

    @pl.when(s < NB)
    def _():
        z2 = _elu(_bn(yz_ref[...], sz_ref[...], g2_ref[...], be2_ref[...]))
        z2s[pl.ds(s * BR, BR), :] = z2
        e = jnp.sum(jnp.exp(z2), axis=0, keepdims=True)
        e = jnp.concatenate([e, jnp.zeros((7, DT), jnp.float32)], axis=0)
        _acc_add(esa, e, s == 0)
        _acc_max(m2a, _segmax(z2, _mask_of(batch_ref[...])), s == 0)

    @pl.when(s >= NB)
    def _():
        i = s - NB
        z2 = z2s[pl.ds(i * BR, BR), :]
        z_ref[...] = zp_ref[...] + (jnp.exp(z2) / esa[0:1, :]) * z2

    @pl.when(s == 2 * NB - 1)
    def _():
        ot_ref[...] = m0_ref[...] + m1_ref[...] + m2a[...]


def _layer4(yz2, sz2, g2, be2, batch2, zp, m0, m1):
    return pl.pallas_call(
        _k4,
        grid=(2 * NB,),
        in_specs=[
            _blkA(DT), _full((8, DT)), _full((1, DT)), _full((1, DT)),
            pl.BlockSpec((BR, 1), lambda s: (jnp.minimum(s, NB - 1), 0)),
            _blkB(DT), _full((DT, G)), _full((DT, G)),
        ],
        out_specs=[_blkB(DT), _full((DT, G))],
        out_shape=[
            jax.ShapeDtypeStruct((N, DT), jnp.float32),
            jax.ShapeDtypeStruct((DT, G), jnp.float32),
        ],
        scratch_shapes=[
            pltpu.VMEM((N, DT), jnp.float32),
            pltpu.VMEM((8, DT), jnp.float32),
            pltpu.VMEM((DT, G), jnp.float32),
        ],
    )(yz2, sz2, g2, be2, batch2, zp, m0, m1)


def kernel(x, edge_index, batch, W0, b0, g0, be0, Wc1, bc1, gc1, bec1,
           W1, b1, g1, be1, Wc2, bc2, gc2, bec2, W2, b2, g2, be2):
    src_r = edge_index[0].reshape(NW * NCH, CH)
    dst_r = edge_index[1].reshape(NW * NCH, CH)
    zeros = jnp.zeros((N, D), jnp.float32)
    batch2 = batch.reshape(N, 1)
    r2 = lambda a: a.reshape(1, -1)

    agg1p = _sc_segment_sum(x, src_r, dst_r, zeros)
    z0, x1, yz1, sz1, m0 = _layer1(x, agg1p, W0, r2(b0), Wc1, r2(bc1),
                                   r2(g0), r2(be0), r2(gc1), r2(bec1),
                                   W1, r2(b1), batch2)
    agg2p = _sc_segment_sum(x1, src_r, dst_r, zeros)
    Zp, m1 = _layer2(yz1, sz1, r2(g1), r2(be1), batch2, z0)
    x2, yz2, sz2 = _layer3(x1, agg2p, Wc2, r2(bc2), r2(gc2), r2(bec2),
                           W2, r2(b2))
    Z, out_t = _layer4(yz2, sz2, r2(g2), r2(be2), batch2, Zp, m0, m1)
    return out_t.T, Z, x2
